# Initial kernel scaffold; baseline (speedup 1.0000x reference)
import functools

import jax
import jax.numpy as jnp
from jax import lax
from jax.experimental import pallas as pl
from jax.experimental.pallas import tpu as pltpu
from jax.experimental.pallas import tpu_sc as plsc

N = 10000
E = 320000
D = 128

NC = 2
NS = 16
NW = NC * NS
EPW = E // NW
CHUNK = 80
NCHUNK = EPW // CHUNK
ROWS_PER_TILE = N // NS


def _sc_agg_body(src_hbm, dst_hbm, x_hbm, zeros_hbm, out_hbm,
                 src_v, dst_v, rows_v, acc_sh, sem):
    c = lax.axis_index("c")
    s = lax.axis_index("s")
    wid = c * NS + s

    pltpu.sync_copy(zeros_hbm.at[pl.ds(s * ROWS_PER_TILE, ROWS_PER_TILE)],
                    acc_sh.at[pl.ds(s * ROWS_PER_TILE, ROWS_PER_TILE)])
    pltpu.sync_copy(src_hbm.at[pl.ds(wid * NCHUNK, NCHUNK)], src_v)
    pltpu.sync_copy(dst_hbm.at[pl.ds(wid * NCHUNK, NCHUNK)], dst_v)
    plsc.subcore_barrier()

    def body(j, carry):
        pltpu.async_copy(x_hbm.at[src_v.at[j]], rows_v, sem).wait()
        pltpu.sync_copy(rows_v, acc_sh.at[dst_v.at[j]], add=True)
        return carry

    lax.fori_loop(0, NCHUNK, body, 0)
    plsc.subcore_barrier()

    pltpu.sync_copy(acc_sh.at[pl.ds(s * ROWS_PER_TILE, ROWS_PER_TILE)],
                    out_hbm.at[c].at[pl.ds(s * ROWS_PER_TILE, ROWS_PER_TILE)])


@jax.jit
def _sc_agg(src2d, dst2d, x, zeros):
    mesh = plsc.VectorSubcoreMesh(core_axis_name="c", subcore_axis_name="s",
                                  num_cores=NC, num_subcores=NS)
    f = pl.kernel(
        _sc_agg_body,
        out_type=jax.ShapeDtypeStruct((NC, N, D), jnp.float32),
        mesh=mesh,
        scratch_types=[
            pltpu.VMEM((NCHUNK, CHUNK), jnp.int32),
            pltpu.VMEM((NCHUNK, CHUNK), jnp.int32),
            pltpu.VMEM((CHUNK, D), jnp.float32),
            pltpu.VMEM_SHARED((N, D), jnp.float32),
            pltpu.SemaphoreType.DMA,
        ],
    )
    return f(src2d, dst2d, x, zeros)


def _tc_mlp_body(x_ref, p_ref, W1_ref, b1_ref, W2_ref, b2_ref, W3_ref, b3_ref,
                 gamma_ref, beta_ref, o_ref):
    h = x_ref[...] + p_ref[0] + p_ref[1]
    h = jax.nn.sigmoid(
        jnp.dot(h, W1_ref[...], preferred_element_type=jnp.float32)
        + b1_ref[...])
    h = jax.nn.sigmoid(
        jnp.dot(h, W2_ref[...], preferred_element_type=jnp.float32)
        + b2_ref[...])
    h = (jnp.dot(h, W3_ref[...], preferred_element_type=jnp.float32)
         + b3_ref[...])
    h = jnp.where(h >= 0, h, 0.01 * h)
    mean = jnp.mean(h, axis=0, keepdims=True)
    var = jnp.mean(h * h, axis=0, keepdims=True) - mean * mean
    o_ref[...] = ((h - mean) * jax.lax.rsqrt(var + 1e-5) * gamma_ref[...]
                  + beta_ref[...])


@jax.jit
def _tc_mlp(x, partials, W1, b1, W2, b2, W3, b3, gamma, beta):
    return pl.pallas_call(
        _tc_mlp_body,
        out_shape=jax.ShapeDtypeStruct((N, D), jnp.float32),
    )(x, partials, W1, b1.reshape(1, -1), W2, b2.reshape(1, -1),
      W3, b3.reshape(1, -1), gamma.reshape(1, -1), beta.reshape(1, -1))


def kernel(x, edge_index, W1, b1, W2, b2, W3, b3, gamma, beta):
    src2d = edge_index[0].reshape(NW * NCHUNK, CHUNK)
    dst2d = edge_index[1].reshape(NW * NCHUNK, CHUNK)
    zeros = jnp.zeros((N, D), jnp.float32)
    partials = _sc_agg(src2d, dst2d, x, zeros)
    h = _tc_mlp(x, partials, W1, b1, W2, b2, W3, b3, gamma, beta)
    return (h, edge_index)


# speedup vs baseline: 6.6537x; 6.6537x over previous
import jax
import jax.numpy as jnp
from jax import lax
from jax.experimental import pallas as pl
from jax.experimental.pallas import tpu as pltpu
from jax.experimental.pallas import tpu_sc as plsc

N = 10000
E = 320000
D = 128

NC = 2
NS = 16
NW = NC * NS
EPW = E // NW
CHUNK = 80
NCHUNK = EPW // CHUNK
ACC_N = 10240
RPT = ACC_N // NS


def _sc_agg_body(src_hbm, dst_hbm, x_hbm, zeros_hbm, out_hbm,
                 src_v, dst_v, rows_v, acc_sh, sem):
    c = lax.axis_index("c")
    s = lax.axis_index("s")
    wid = c * NS + s

    pltpu.sync_copy(zeros_hbm.at[s], acc_sh.at[pl.ds(s * RPT, RPT)])
    pltpu.sync_copy(src_hbm.at[wid], src_v)
    pltpu.sync_copy(dst_hbm.at[wid], dst_v)
    plsc.subcore_barrier()

    def body(j, carry):
        pltpu.async_copy(x_hbm.at[src_v.at[j]], rows_v, sem).wait()
        pltpu.sync_copy(rows_v, acc_sh.at[dst_v.at[j]], add=True)
        return carry

    lax.fori_loop(0, NCHUNK, body, 0)
    plsc.subcore_barrier()

    pltpu.sync_copy(acc_sh.at[pl.ds(s * RPT, RPT)], out_hbm.at[c, s])


@jax.jit
def _sc_agg(src3d, dst3d, x, zeros):
    mesh = plsc.VectorSubcoreMesh(core_axis_name="c", subcore_axis_name="s",
                                  num_cores=NC, num_subcores=NS)
    f = pl.kernel(
        _sc_agg_body,
        out_type=jax.ShapeDtypeStruct((NC, NS, RPT, D), jnp.float32),
        mesh=mesh,
        scratch_types=[
            pltpu.VMEM((NCHUNK, CHUNK), jnp.int32),
            pltpu.VMEM((NCHUNK, CHUNK), jnp.int32),
            pltpu.VMEM((CHUNK, D), jnp.float32),
            pltpu.VMEM_SHARED((ACC_N, D), jnp.float32),
            pltpu.SemaphoreType.DMA,
        ],
    )
    return f(src3d, dst3d, x, zeros)


def _tc_mlp_body(x_ref, p_ref, W1_ref, b1_ref, W2_ref, b2_ref, W3_ref, b3_ref,
                 gamma_ref, beta_ref, o_ref):
    h = x_ref[...] + p_ref[0] + p_ref[1]
    h = jax.nn.sigmoid(
        jnp.dot(h, W1_ref[...], preferred_element_type=jnp.float32)
        + b1_ref[...])
    h = jax.nn.sigmoid(
        jnp.dot(h, W2_ref[...], preferred_element_type=jnp.float32)
        + b2_ref[...])
    h = (jnp.dot(h, W3_ref[...], preferred_element_type=jnp.float32)
         + b3_ref[...])
    h = jnp.where(h >= 0, h, 0.01 * h)
    mean = jnp.mean(h, axis=0, keepdims=True)
    var = jnp.mean(h * h, axis=0, keepdims=True) - mean * mean
    o_ref[...] = ((h - mean) * jax.lax.rsqrt(var + 1e-5) * gamma_ref[...]
                  + beta_ref[...])


@jax.jit
def _tc_mlp(x, partials, W1, b1, W2, b2, W3, b3, gamma, beta):
    return pl.pallas_call(
        _tc_mlp_body,
        out_shape=jax.ShapeDtypeStruct((N, D), jnp.float32),
    )(x, partials, W1, b1.reshape(1, -1), W2, b2.reshape(1, -1),
      W3, b3.reshape(1, -1), gamma.reshape(1, -1), beta.reshape(1, -1))


def kernel(x, edge_index, W1, b1, W2, b2, W3, b3, gamma, beta):
    src3d = edge_index[0].reshape(NW, NCHUNK, CHUNK)
    dst3d = edge_index[1].reshape(NW, NCHUNK, CHUNK)
    zeros = jnp.zeros((NS, RPT, D), jnp.float32)
    out4d = _sc_agg(src3d, dst3d, x, zeros)
    partials = out4d.reshape(NC, ACC_N, D)[:, :N]
    h = _tc_mlp(x, partials, W1, b1, W2, b2, W3, b3, gamma, beta)
    return (h, edge_index)
